# baseline (speedup 1.0000x reference)
import functools

import jax
import jax.numpy as jnp
from jax import lax
from jax.experimental import pallas as pl
from jax.experimental.pallas import tpu as pltpu
from jax.experimental.pallas import tpu_sc as plsc

_B = 16384
_D = 128
_V = 512
_NC = 2
_NS = 16
_NW = _NC * _NS
_BPW = _B // _NW
_CH = 128
_NCH = _BPW // _CH
_NB = 4

_BLK = 256
_NBLK = _B // _BLK


def _sc_body(tab, xi, out, idx_v, bufs, *sems):
    gsems = sems[:_NB]
    wsems = sems[_NB:]
    wid = lax.axis_index("s") * _NC + lax.axis_index("c")
    base = wid * _BPW
    pltpu.sync_copy(xi.at[pl.ds(base, _BPW)], idx_v)

    gh = [None] * _NCH
    wh = [None] * _NCH
    for t in range(_NCH):
        b = t % _NB
        if t >= _NB:
            wh[t - _NB].wait()
        gh[t] = pltpu.async_copy(tab.at[idx_v.at[pl.ds(t * _CH, _CH)]],
                                 bufs.at[b], gsems[b])
        d = t - (_NB - 1)
        if d >= 0:
            gh[d].wait()
            wh[d] = pltpu.async_copy(bufs.at[d % _NB],
                                     out.at[pl.ds(base + d * _CH, _CH)],
                                     wsems[d % _NB])
    for d in range(max(_NCH - (_NB - 1), 0), _NCH):
        gh[d].wait()
        wh[d] = pltpu.async_copy(bufs.at[d % _NB],
                                 out.at[pl.ds(base + d * _CH, _CH)],
                                 wsems[d % _NB])
    for d in range(max(_NCH - _NB, 0), _NCH):
        wh[d].wait()


_sc_gather = functools.partial(
    pl.kernel,
    mesh=plsc.VectorSubcoreMesh(core_axis_name="c", subcore_axis_name="s"),
    out_type=jax.ShapeDtypeStruct((_B, _D), jnp.float32),
    scratch_types=[
        pltpu.VMEM((_BPW,), jnp.int32),
        pltpu.VMEM((_NB, _CH, _D), jnp.float32),
    ] + [pltpu.SemaphoreType.DMA] * (2 * _NB),
)(_sc_body)


def _tc_body(idx_ref, tab_ref, out_ref):
    idx = idx_ref[0, 0, :].reshape(_BLK, 1)
    vids = lax.broadcasted_iota(jnp.int32, (_BLK, _V), 1)
    onehot = (vids == idx).astype(jnp.float32)
    out_ref[...] = jax.lax.dot_general(
        onehot, tab_ref[...],
        dimension_numbers=(((1,), (0,)), ((), ())),
        preferred_element_type=jnp.float32)


_tc_lookup = pl.pallas_call(
    _tc_body,
    grid=(_NBLK,),
    in_specs=[
        pl.BlockSpec((1, 1, _BLK), lambda i: (i, 0, 0)),
        pl.BlockSpec((_V, _D), lambda i: (0, 0)),
    ],
    out_specs=pl.BlockSpec((_BLK, _D), lambda i: (i, 0)),
    out_shape=jax.ShapeDtypeStruct((_B, _D), jnp.float32),
)


@jax.jit
def kernel(x, y, row_table, col_table):
    xf = x.reshape(-1).astype(jnp.int32)
    y3 = y.astype(jnp.int32).reshape(_NBLK, 1, _BLK)
    row = _sc_gather(row_table, xf)
    col = _tc_lookup(y3, col_table)
    return row, col


# speedup vs baseline: 1.3149x; 1.3149x over previous
import functools

import jax
import jax.numpy as jnp
from jax import lax
from jax.experimental import pallas as pl
from jax.experimental.pallas import tpu as pltpu
from jax.experimental.pallas import tpu_sc as plsc

_B = 16384
_D = 128
_V = 512
_NC = 2
_NS = 16
_NW = _NC * _NS
_BPW = _B // _NW
_CH = 128
_NCH = _BPW // _CH
_NB = 4

_BLK = 512
_NBLK = _B // _BLK


def _sc_body(tab, xi, out, idx_v, bufs, *sems):
    gsems = sems[:_NB]
    wsems = sems[_NB:]
    wid = lax.axis_index("s") * _NC + lax.axis_index("c")
    base = wid * _BPW
    pltpu.sync_copy(xi.at[pl.ds(base, _BPW)], idx_v)

    gh = [None] * _NCH
    wh = [None] * _NCH
    for t in range(_NCH):
        b = t % _NB
        if t >= _NB:
            wh[t - _NB].wait()
        gh[t] = pltpu.async_copy(tab.at[idx_v.at[pl.ds(t * _CH, _CH)]],
                                 bufs.at[b], gsems[b])
        d = t - (_NB - 1)
        if d >= 0:
            gh[d].wait()
            wh[d] = pltpu.async_copy(bufs.at[d % _NB],
                                     out.at[pl.ds(base + d * _CH, _CH)],
                                     wsems[d % _NB])
    for d in range(max(_NCH - (_NB - 1), 0), _NCH):
        gh[d].wait()
        wh[d] = pltpu.async_copy(bufs.at[d % _NB],
                                 out.at[pl.ds(base + d * _CH, _CH)],
                                 wsems[d % _NB])
    for d in range(max(_NCH - _NB, 0), _NCH):
        wh[d].wait()


_sc_gather = functools.partial(
    pl.kernel,
    mesh=plsc.VectorSubcoreMesh(core_axis_name="c", subcore_axis_name="s"),
    out_type=jax.ShapeDtypeStruct((_B, _D), jnp.float32),
    scratch_types=[
        pltpu.VMEM((_BPW,), jnp.int32),
        pltpu.VMEM((_NB, _CH, _D), jnp.float32),
    ] + [pltpu.SemaphoreType.DMA] * (2 * _NB),
)(_sc_body)


def _tc_body(idx_ref, tab_ref, out_ref, hi_s, lo_s):
    @pl.when(pl.program_id(0) == 0)
    def _():
        tab = tab_ref[...]
        hi = tab.astype(jnp.bfloat16)
        hi_s[...] = hi
        lo_s[...] = (tab - hi.astype(jnp.float32)).astype(jnp.bfloat16)

    idx = idx_ref[0, 0, :].reshape(_BLK, 1)
    vids = lax.broadcasted_iota(jnp.int32, (_BLK, _V), 1)
    onehot = (vids == idx).astype(jnp.float32).astype(jnp.bfloat16)
    dn = (((1,), (0,)), ((), ()))
    acc = jax.lax.dot_general(onehot, hi_s[...], dimension_numbers=dn,
                              preferred_element_type=jnp.float32)
    acc += jax.lax.dot_general(onehot, lo_s[...], dimension_numbers=dn,
                               preferred_element_type=jnp.float32)
    out_ref[...] = acc


_tc_lookup = pl.pallas_call(
    _tc_body,
    grid=(_NBLK,),
    in_specs=[
        pl.BlockSpec((1, 1, _BLK), lambda i: (i, 0, 0)),
        pl.BlockSpec((_V, _D), lambda i: (0, 0)),
    ],
    out_specs=pl.BlockSpec((_BLK, _D), lambda i: (i, 0)),
    out_shape=jax.ShapeDtypeStruct((_B, _D), jnp.float32),
    scratch_shapes=[
        pltpu.VMEM((_V, _D), jnp.bfloat16),
        pltpu.VMEM((_V, _D), jnp.bfloat16),
    ],
)


@jax.jit
def kernel(x, y, row_table, col_table):
    xf = x.reshape(-1).astype(jnp.int32)
    y3 = y.astype(jnp.int32).reshape(_NBLK, 1, _BLK)
    row = _sc_gather(row_table, xf)
    col = _tc_lookup(y3, col_table)
    return row, col


# speedup vs baseline: 1.3159x; 1.0007x over previous
import functools

import jax
import jax.numpy as jnp
from jax import lax
from jax.experimental import pallas as pl
from jax.experimental.pallas import tpu as pltpu
from jax.experimental.pallas import tpu_sc as plsc

_B = 16384
_D = 128
_V = 512
_NC = 2
_NS = 16
_NW = _NC * _NS
_BPW = _B // _NW
_CH = 128
_NCH = _BPW // _CH
_NB = 4

_BLK = 512
_NBLK = _B // _BLK


def _sc_body(tab, xi, out, idx_v, bufs, *sems):
    gsems = sems[:_NB]
    wsems = sems[_NB:]
    wid = lax.axis_index("s") * _NC + lax.axis_index("c")
    base = wid * _BPW
    pltpu.sync_copy(xi.at[pl.ds(base, _BPW)], idx_v)

    gh = [None] * _NCH
    wh = [None] * _NCH
    for t in range(_NCH):
        b = t % _NB
        if t >= _NB:
            wh[t - _NB].wait()
        gh[t] = pltpu.async_copy(tab.at[idx_v.at[pl.ds(t * _CH, _CH)]],
                                 bufs.at[b], gsems[b])
        d = t - (_NB - 1)
        if d >= 0:
            gh[d].wait()
            wh[d] = pltpu.async_copy(bufs.at[d % _NB],
                                     out.at[pl.ds(base + d * _CH, _CH)],
                                     wsems[d % _NB])
    for d in range(max(_NCH - (_NB - 1), 0), _NCH):
        gh[d].wait()
        wh[d] = pltpu.async_copy(bufs.at[d % _NB],
                                 out.at[pl.ds(base + d * _CH, _CH)],
                                 wsems[d % _NB])
    for d in range(max(_NCH - _NB, 0), _NCH):
        wh[d].wait()


_sc_gather = functools.partial(
    pl.kernel,
    mesh=plsc.VectorSubcoreMesh(core_axis_name="c", subcore_axis_name="s"),
    out_type=jax.ShapeDtypeStruct((_B, _D), jnp.float32),
    scratch_types=[
        pltpu.VMEM((_BPW,), jnp.int32),
        pltpu.VMEM((_NB, _CH, _D), jnp.float32),
    ] + [pltpu.SemaphoreType.DMA] * (2 * _NB),
)(_sc_body)


def _tc_body(idx_ref, tab_ref, out_ref, hi_s, lo_s):
    @pl.when(pl.program_id(0) == 0)
    def _():
        tab = tab_ref[...]
        hi = tab.astype(jnp.bfloat16)
        hi_s[...] = hi
        lo_s[...] = (tab - hi.astype(jnp.float32)).astype(jnp.bfloat16)

    idx = idx_ref[0, 0, :].astype(jnp.int16).reshape(_BLK, 1)
    vids = lax.broadcasted_iota(jnp.int16, (_BLK, _V), 1)
    onehot = jnp.where(vids == idx, jnp.bfloat16(1), jnp.bfloat16(0))
    dn = (((1,), (0,)), ((), ()))
    acc = jax.lax.dot_general(onehot, hi_s[...], dimension_numbers=dn,
                              preferred_element_type=jnp.float32)
    acc += jax.lax.dot_general(onehot, lo_s[...], dimension_numbers=dn,
                               preferred_element_type=jnp.float32)
    out_ref[...] = acc


_tc_lookup = pl.pallas_call(
    _tc_body,
    grid=(_NBLK,),
    in_specs=[
        pl.BlockSpec((1, 1, _BLK), lambda i: (i, 0, 0)),
        pl.BlockSpec((_V, _D), lambda i: (0, 0)),
    ],
    out_specs=pl.BlockSpec((_BLK, _D), lambda i: (i, 0)),
    out_shape=jax.ShapeDtypeStruct((_B, _D), jnp.float32),
    scratch_shapes=[
        pltpu.VMEM((_V, _D), jnp.bfloat16),
        pltpu.VMEM((_V, _D), jnp.bfloat16),
    ],
)


@jax.jit
def kernel(x, y, row_table, col_table):
    xf = x.reshape(-1).astype(jnp.int32)
    y3 = y.astype(jnp.int32).reshape(_NBLK, 1, _BLK)
    row = _sc_gather(row_table, xf)
    col = _tc_lookup(y3, col_table)
    return row, col


# speedup vs baseline: 1.5047x; 1.1435x over previous
import functools

import jax
import jax.numpy as jnp
from jax import lax
from jax.experimental import pallas as pl
from jax.experimental.pallas import tpu as pltpu
from jax.experimental.pallas import tpu_sc as plsc

_B = 16384
_D = 128
_V = 512
_NC = 2
_NS = 16
_NW = _NC * _NS
_BPW = _B // _NW
_CH = 128
_NCH = _BPW // _CH
_NB = 4

_BLK = 1024
_NBLK = _B // _BLK


def _sc_body(tab, xi, out, idx_v, bufs, *sems):
    gsems = sems[:_NB]
    wsems = sems[_NB:]
    wid = lax.axis_index("s") * _NC + lax.axis_index("c")
    base = wid * _BPW
    pltpu.sync_copy(xi.at[pl.ds(base, _BPW)], idx_v)

    gh = [None] * _NCH
    wh = [None] * _NCH
    for t in range(_NCH):
        b = t % _NB
        if t >= _NB:
            wh[t - _NB].wait()
        gh[t] = pltpu.async_copy(tab.at[idx_v.at[pl.ds(t * _CH, _CH)]],
                                 bufs.at[b], gsems[b])
        d = t - (_NB - 1)
        if d >= 0:
            gh[d].wait()
            wh[d] = pltpu.async_copy(bufs.at[d % _NB],
                                     out.at[pl.ds(base + d * _CH, _CH)],
                                     wsems[d % _NB])
    for d in range(max(_NCH - (_NB - 1), 0), _NCH):
        gh[d].wait()
        wh[d] = pltpu.async_copy(bufs.at[d % _NB],
                                 out.at[pl.ds(base + d * _CH, _CH)],
                                 wsems[d % _NB])
    for d in range(max(_NCH - _NB, 0), _NCH):
        wh[d].wait()


_sc_gather = functools.partial(
    pl.kernel,
    mesh=plsc.VectorSubcoreMesh(core_axis_name="c", subcore_axis_name="s"),
    out_type=jax.ShapeDtypeStruct((_B, _D), jnp.float32),
    scratch_types=[
        pltpu.VMEM((_BPW,), jnp.int32),
        pltpu.VMEM((_NB, _CH, _D), jnp.float32),
    ] + [pltpu.SemaphoreType.DMA] * (2 * _NB),
)(_sc_body)


def _tc_body(idx_ref, tab_ref, out_ref, hi_s, lo_s):
    @pl.when(pl.program_id(0) == 0)
    def _():
        tab = tab_ref[...]
        hi = tab.astype(jnp.bfloat16)
        hi_s[...] = hi
        lo_s[...] = (tab - hi.astype(jnp.float32)).astype(jnp.bfloat16)

    idx = idx_ref[0, 0, :].astype(jnp.int16).reshape(_BLK, 1)
    vids = lax.broadcasted_iota(jnp.int16, (_BLK, _V), 1)
    onehot = jnp.where(vids == idx, jnp.bfloat16(1), jnp.bfloat16(0))
    dn = (((1,), (0,)), ((), ()))
    acc = jax.lax.dot_general(onehot, hi_s[...], dimension_numbers=dn,
                              preferred_element_type=jnp.float32)
    acc += jax.lax.dot_general(onehot, lo_s[...], dimension_numbers=dn,
                               preferred_element_type=jnp.float32)
    out_ref[...] = acc


_tc_lookup = pl.pallas_call(
    _tc_body,
    grid=(_NBLK,),
    in_specs=[
        pl.BlockSpec((1, 1, _BLK), lambda i: (i, 0, 0)),
        pl.BlockSpec((_V, _D), lambda i: (0, 0)),
    ],
    out_specs=pl.BlockSpec((_BLK, _D), lambda i: (i, 0)),
    out_shape=jax.ShapeDtypeStruct((_B, _D), jnp.float32),
    scratch_shapes=[
        pltpu.VMEM((_V, _D), jnp.bfloat16),
        pltpu.VMEM((_V, _D), jnp.bfloat16),
    ],
)


@jax.jit
def kernel(x, y, row_table, col_table):
    xf = x.reshape(-1).astype(jnp.int32)
    y3 = y.astype(jnp.int32).reshape(_NBLK, 1, _BLK)
    row = _sc_gather(row_table, xf)
    col = _tc_lookup(y3, col_table)
    return row, col
